# baseline (speedup 1.0000x reference)
import functools

import jax
import jax.numpy as jnp
from jax import lax
from jax.experimental import pallas as pl
from jax.experimental.pallas import tpu as pltpu
from jax.experimental.pallas import tpu_sc as plsc

EMB = 128
CH = 128


@functools.lru_cache(maxsize=None)
def _build_sc_gather(b_flat: int):
    info = plsc.get_sparse_core_info()
    nc, ns, nl = info.num_cores, info.num_subcores, info.num_lanes
    nw = nc * ns
    rows_per_w = b_flat // nw
    n_ch = rows_per_w // CH
    assert rows_per_w % CH == 0
    mesh = plsc.VectorSubcoreMesh(core_axis_name="c", subcore_axis_name="s")

    @functools.partial(
        pl.kernel,
        mesh=mesh,
        out_type=jax.ShapeDtypeStruct((b_flat, EMB), jnp.float32),
        scratch_types=[
            pltpu.VMEM((CH,), jnp.int32),
            pltpu.VMEM((CH, EMB), jnp.float32),
            pltpu.SemaphoreType.DMA,
        ],
    )
    def k(fused_hbm, x_hbm, out_hbm, idx_v, rows_v, sem):
        wid = lax.axis_index("s") * nc + lax.axis_index("c")
        base = wid * rows_per_w
        lane = lax.iota(jnp.int32, nl) % 4
        offs = jnp.where(lane == 0, 0, 2 * lane + 2)
        for c in range(n_ch):
            lo = base + c * CH
            pltpu.sync_copy(x_hbm.at[pl.ds(lo, CH)], idx_v)
            for j in range(CH // nl):
                sl = pl.ds(j * nl, nl)
                idx_v[sl] = idx_v[sl] + offs
            pltpu.async_copy(fused_hbm.at[idx_v], rows_v, sem).wait()
            pltpu.sync_copy(rows_v, out_hbm.at[pl.ds(lo, CH)])

    return k


def kernel(x, street_table, action_table, position_table, order_table):
    batch = x.shape[0]
    fused = jnp.concatenate(
        (
            street_table,
            position_table + order_table[0],
            position_table + order_table[1],
            action_table,
        ),
        axis=0,
    )
    x_flat = x.astype(jnp.int32).reshape(-1)
    out = _build_sc_gather(4 * batch)(fused, x_flat)
    return out.reshape(batch, 4 * EMB)


# speedup vs baseline: 1.0126x; 1.0126x over previous
import functools

import jax
import jax.numpy as jnp
from jax import lax
from jax.experimental import pallas as pl
from jax.experimental.pallas import tpu as pltpu
from jax.experimental.pallas import tpu_sc as plsc

EMB = 128
CH = 128
NB = 4


@functools.lru_cache(maxsize=None)
def _build_sc_gather(b_flat: int):
    info = plsc.get_sparse_core_info()
    nc, ns, nl = info.num_cores, info.num_subcores, info.num_lanes
    nw = nc * ns
    rows_per_w = b_flat // nw
    n_ch = rows_per_w // CH
    assert rows_per_w % CH == 0 and n_ch >= NB
    mesh = plsc.VectorSubcoreMesh(core_axis_name="c", subcore_axis_name="s")

    @functools.partial(
        pl.kernel,
        mesh=mesh,
        out_type=jax.ShapeDtypeStruct((b_flat, EMB), jnp.float32),
        scratch_types=[
            pltpu.VMEM((rows_per_w,), jnp.int32),
            pltpu.VMEM((NB, CH, EMB), jnp.float32),
        ]
        + [pltpu.SemaphoreType.DMA] * (2 * NB),
    )
    def k(fused_hbm, x_hbm, out_hbm, idx_v, rows_v, *sems):
        sem_g, sem_o = sems[:NB], sems[NB:]
        wid = lax.axis_index("s") * nc + lax.axis_index("c")
        base = wid * rows_per_w
        pltpu.sync_copy(x_hbm.at[pl.ds(base, rows_per_w)], idx_v)
        lane = lax.iota(jnp.int32, nl) % 4
        offs = jnp.where(lane == 0, 0, 2 * lane + 2)
        for j in range(rows_per_w // nl):
            sl = pl.ds(j * nl, nl)
            idx_v[sl] = idx_v[sl] + offs

        def g_start(c):
            b = c % NB
            return pltpu.async_copy(
                fused_hbm.at[idx_v.at[pl.ds(c * CH, CH)]], rows_v.at[b], sem_g[b]
            )

        def o_start(c):
            b = c % NB
            return pltpu.async_copy(
                rows_v.at[b], out_hbm.at[pl.ds(base + c * CH, CH)], sem_o[b]
            )

        gh, oh = {}, {}
        for c in range(NB):
            gh[c] = g_start(c)
        pending = []
        for c in range(n_ch):
            gh[c].wait()
            oh[c] = o_start(c)
            pending.append(c)
            p = c - (NB - 1)
            if p >= 0 and p + NB < n_ch:
                oh[p].wait()
                pending.remove(p)
                gh[p + NB] = g_start(p + NB)
        for c in pending:
            oh[c].wait()

    return k


def kernel(x, street_table, action_table, position_table, order_table):
    batch = x.shape[0]
    fused = jnp.concatenate(
        (
            street_table,
            position_table + order_table[0],
            position_table + order_table[1],
            action_table,
        ),
        axis=0,
    )
    x_flat = x.astype(jnp.int32).reshape(-1)
    out = _build_sc_gather(4 * batch)(fused, x_flat)
    return out.reshape(batch, 4 * EMB)


# speedup vs baseline: 4.4263x; 4.3710x over previous
import functools

import jax
import jax.numpy as jnp
from jax import lax
from jax.experimental import pallas as pl
from jax.experimental.pallas import tpu as pltpu
from jax.experimental.pallas import tpu_sc as plsc

EMB = 128
CH = 128
NB = 2

_GATHER_DNUMS = lax.GatherDimensionNumbers(
    offset_dims=(), collapsed_slice_dims=(0,), start_index_map=(0,)
)


@functools.lru_cache(maxsize=None)
def _build_sc_compute(b_flat: int):
    info = plsc.get_sparse_core_info()
    nc, ns, nl = info.num_cores, info.num_subcores, info.num_lanes
    nw = nc * ns
    rows_per_w = b_flat // nw
    n_ch = rows_per_w // CH
    n_outer = n_ch // NB
    assert rows_per_w % CH == 0 and n_ch % NB == 0
    mesh = plsc.VectorSubcoreMesh(core_axis_name="c", subcore_axis_name="s")

    @functools.partial(
        pl.kernel,
        mesh=mesh,
        out_type=jax.ShapeDtypeStruct((b_flat * EMB,), jnp.float32),
        scratch_types=[
            pltpu.VMEM((640,), jnp.float32),
            pltpu.VMEM((rows_per_w,), jnp.int32),
            pltpu.VMEM((NB * CH * EMB,), jnp.float32),
        ]
        + [pltpu.SemaphoreType.DMA] * NB,
    )
    def k(coef_hbm, x_hbm, out_hbm, coef_v, xi_v, rows_v, *sem_o):
        wid = lax.axis_index("s") * nc + lax.axis_index("c")
        base = wid * rows_per_w
        pltpu.sync_copy(coef_hbm, coef_v)
        pltpu.sync_copy(x_hbm.at[pl.ds(base, rows_per_w)], xi_v)
        st1 = [coef_v[pl.ds(j * nl, nl)] for j in range(8)]
        dP = [coef_v[pl.ds(128 + j * nl, nl)] for j in range(8)]
        bH = [coef_v[pl.ds(256 + j * nl, nl)] for j in range(8)]
        bV = [coef_v[pl.ds(384 + j * nl, nl)] for j in range(8)]
        ac1 = [coef_v[pl.ds(512 + j * nl, nl)] for j in range(8)]

        def out_copy(c, b, start):
            src = rows_v.at[pl.ds(b * CH * EMB, CH * EMB)]
            dst = out_hbm.at[pl.ds((base + c * CH) * EMB, CH * EMB)]
            if start:
                return pltpu.async_copy(src, dst, sem_o[b])
            return pltpu.make_async_copy(src, dst, sem_o[b]).wait()

        def body(s, carry):
            for b in range(NB):
                c = s * NB + b

                @pl.when(s > 0)
                def _():
                    out_copy(c - NB, b, start=False)

                for g in range(CH // nl):
                    x16 = xi_v[pl.ds(c * CH + g * nl, nl)].astype(jnp.float32)
                    for kk in range(nl):
                        r = g * nl + kk
                        m = lax.gather(
                            x16,
                            jnp.full((nl, 1), kk, jnp.int32),
                            _GATHER_DNUMS,
                            slice_sizes=(1,),
                            mode=lax.GatherScatterMode.PROMISE_IN_BOUNDS,
                        )
                        seg = r % 4
                        off = (b * CH + g * nl + kk) * EMB
                        for j in range(8):
                            if seg == 0:
                                row = m * st1[j]
                            elif seg == 1:
                                row = bH[j] + m * dP[j]
                            elif seg == 2:
                                row = bV[j] + m * dP[j]
                            else:
                                row = m * ac1[j]
                            rows_v[pl.ds(off + j * nl, nl)] = row
                out_copy(c, b, start=True)
            return carry

        lax.fori_loop(0, n_outer, body, 0)
        for b in range(NB):
            out_copy(n_ch - NB + b, b, start=False)

    return k


def kernel(x, street_table, action_table, position_table, order_table):
    batch = x.shape[0]
    coef = jnp.concatenate(
        (
            street_table[1],
            position_table[1] - position_table[0],
            position_table[0] + order_table[0],
            position_table[0] + order_table[1],
            action_table[1],
        )
    )
    x_flat = x.astype(jnp.int32).reshape(-1)
    out = _build_sc_compute(4 * batch)(coef, x_flat)
    return out.reshape(batch, 4 * EMB)


# speedup vs baseline: 5.2121x; 1.1775x over previous
import functools

import jax
import jax.numpy as jnp
from jax import lax
from jax.experimental import pallas as pl
from jax.experimental.pallas import tpu as pltpu
from jax.experimental.pallas import tpu_sc as plsc

EMB = 128
CH = 256
NB = 2

_GATHER_DNUMS = lax.GatherDimensionNumbers(
    offset_dims=(), collapsed_slice_dims=(0,), start_index_map=(0,)
)


@functools.lru_cache(maxsize=None)
def _build_sc_compute(b_flat: int):
    info = plsc.get_sparse_core_info()
    nc, ns, nl = info.num_cores, info.num_subcores, info.num_lanes
    nw = nc * ns
    rows_per_w = b_flat // nw
    n_ch = rows_per_w // CH
    n_outer = n_ch // NB
    assert rows_per_w % CH == 0 and n_ch % NB == 0
    mesh = plsc.VectorSubcoreMesh(core_axis_name="c", subcore_axis_name="s")

    @functools.partial(
        pl.kernel,
        mesh=mesh,
        out_type=jax.ShapeDtypeStruct((b_flat * EMB,), jnp.float32),
        scratch_types=[
            pltpu.VMEM((640,), jnp.float32),
            pltpu.VMEM((rows_per_w,), jnp.int32),
            pltpu.VMEM((NB * CH * EMB,), jnp.float32),
        ]
        + [pltpu.SemaphoreType.DMA] * NB,
    )
    def k(coef_hbm, x_hbm, out_hbm, coef_v, xi_v, rows_v, *sem_o):
        wid = lax.axis_index("s") * nc + lax.axis_index("c")
        base = wid * rows_per_w
        pltpu.sync_copy(coef_hbm, coef_v)
        pltpu.sync_copy(x_hbm.at[pl.ds(base, rows_per_w)], xi_v)
        st1 = [coef_v[pl.ds(j * nl, nl)] for j in range(8)]
        dP = [coef_v[pl.ds(128 + j * nl, nl)] for j in range(8)]
        bH = [coef_v[pl.ds(256 + j * nl, nl)] for j in range(8)]
        bV = [coef_v[pl.ds(384 + j * nl, nl)] for j in range(8)]
        ac1 = [coef_v[pl.ds(512 + j * nl, nl)] for j in range(8)]

        def out_copy(c, b, start):
            src = rows_v.at[pl.ds(b * CH * EMB, CH * EMB)]
            dst = out_hbm.at[pl.ds((base + c * CH) * EMB, CH * EMB)]
            if start:
                return pltpu.async_copy(src, dst, sem_o[b])
            return pltpu.make_async_copy(src, dst, sem_o[b]).wait()

        def group(c, b, g):
            x16 = xi_v[pl.ds(c * CH + g * nl, nl)].astype(jnp.float32)
            for kk in range(nl):
                m = lax.gather(
                    x16,
                    jnp.full((nl, 1), kk, jnp.int32),
                    _GATHER_DNUMS,
                    slice_sizes=(1,),
                    mode=lax.GatherScatterMode.PROMISE_IN_BOUNDS,
                )
                seg = kk % 4
                off = (b * CH + g * nl + kk) * EMB
                for j in range(8):
                    if seg == 0:
                        row = m * st1[j]
                    elif seg == 1:
                        row = bH[j] + m * dP[j]
                    elif seg == 2:
                        row = bV[j] + m * dP[j]
                    else:
                        row = m * ac1[j]
                    rows_v[pl.ds(off + j * nl, nl)] = row

        def body(s, carry):
            for b in range(NB):
                c = s * NB + b

                @pl.when(s > 0)
                def _():
                    out_copy(c - NB, b, start=False)

                def gbody(g, carry2):
                    group(c, b, g)
                    return carry2

                lax.fori_loop(0, CH // nl, gbody, 0)
                out_copy(c, b, start=True)
            return carry

        lax.fori_loop(0, n_outer, body, 0)
        for b in range(NB):
            out_copy(n_ch - NB + b, b, start=False)

    return k


def kernel(x, street_table, action_table, position_table, order_table):
    batch = x.shape[0]
    coef = jnp.concatenate(
        (
            street_table[1],
            position_table[1] - position_table[0],
            position_table[0] + order_table[0],
            position_table[0] + order_table[1],
            action_table[1],
        )
    )
    x_flat = x.astype(jnp.int32).reshape(-1)
    out = _build_sc_compute(4 * batch)(coef, x_flat)
    return out.reshape(batch, 4 * EMB)
